# baseline (speedup 1.0000x reference)
import functools

import jax
import jax.numpy as jnp
from jax import lax
from jax.experimental import pallas as pl
from jax.experimental.pallas import tpu as pltpu
from jax.experimental.pallas import tpu_sc as plsc

_N_FIELDS = 26
_VOCAB = 100000
_D = 32
_B = 16384

_NC = 2
_NS = 16
_NW = _NC * _NS
_BPW = _B // _NW
_K = 128
_KCH = _BPW // _K
_ROWS_PER_FIELD = _B // _K


def _sc_body(idx_hbm, table_hbm, out_hbm, idx_v, rows_v, gsem):
    wid = lax.axis_index("s") * _NC + lax.axis_index("c")
    row_base = wid * _KCH

    def field_body(f, carry):
        pltpu.sync_copy(
            idx_hbm.at[pl.ds(f * _ROWS_PER_FIELD + row_base, _KCH)], idx_v
        )
        off = f * _VOCAB

        for j in range(_KCH):
            def add16(i, c, j=j):
                sl = pl.ds(i * 16, 16)
                idx_v[j, sl] = idx_v[j, sl] + off
                return c
            lax.fori_loop(0, _K // 16, add16, 0)

        copies = [
            pltpu.async_copy(
                table_hbm.at[idx_v.at[j]],
                rows_v.at[pl.ds(j * _K, _K)],
                gsem,
            )
            for j in range(_KCH)
        ]
        for c in copies:
            c.wait()

        pltpu.sync_copy(rows_v, out_hbm.at[pl.ds(f * _B + wid * _BPW, _BPW)])
        return carry

    lax.fori_loop(0, _N_FIELDS, field_body, 0)


@functools.partial(jax.jit, static_argnames=())
def kernel(inputs, tables):
    idx2d = inputs[..., 0].reshape(_N_FIELDS * _B // _K, _K)
    table_flat = tables.reshape(_N_FIELDS * _VOCAB, _D)

    mesh = plsc.VectorSubcoreMesh(core_axis_name="c", subcore_axis_name="s")
    run = pl.kernel(
        _sc_body,
        out_type=jax.ShapeDtypeStruct((_N_FIELDS * _B, _D), jnp.float32),
        mesh=mesh,
        scratch_types=[
            pltpu.VMEM((_KCH, _K), jnp.int32),
            pltpu.VMEM((_BPW, _D), jnp.float32),
            pltpu.SemaphoreType.DMA,
        ],
        compiler_params=pltpu.CompilerParams(use_tc_tiling_on_sc=False),
    )
    out_flat = run(idx2d, table_flat)
    return out_flat.reshape(_N_FIELDS, _B, _D)


# speedup vs baseline: 1.0224x; 1.0224x over previous
import functools

import jax
import jax.numpy as jnp
from jax import lax
from jax.experimental import pallas as pl
from jax.experimental.pallas import tpu as pltpu
from jax.experimental.pallas import tpu_sc as plsc

_N_FIELDS = 26
_VOCAB = 100000
_D = 32
_B = 16384

_NC = 2
_NS = 16
_NW = _NC * _NS
_K = 128
_TOTAL = _N_FIELDS * _B
_NROWS = _TOTAL // (_NW * _K)
_SUP = 8
_NSUP = _NROWS // _SUP
_CHUNK = _SUP * _K
_NBUF = 3


def _sc_body(idx_hbm, table_hbm, out_hbm, idx_v, rows_v, gsem, osem):
    wid = lax.axis_index("s") * _NC + lax.axis_index("c")
    row0 = wid * _NROWS
    out0 = wid * _NROWS * _K

    pltpu.sync_copy(idx_hbm.at[pl.ds(row0, _NROWS)], idx_v)

    def add_row(r, c):
        off = ((row0 + r) // (_B // _K)) * _VOCAB
        for k in range(_K // 16):
            sl = pl.ds(k * 16, 16)
            idx_v[r, sl] = idx_v[r, sl] + off
        return c

    lax.fori_loop(0, _NROWS, add_row, 0)

    def fire(s):
        buf = rows_v.at[s % _NBUF]
        sem = gsem.at[s % _NBUF]
        for j in range(_SUP):
            pltpu.async_copy(
                table_hbm.at[idx_v.at[s * _SUP + j]],
                buf.at[pl.ds(j * _K, _K)],
                sem,
            )

    def wait_gathers(s):
        buf = rows_v.at[s % _NBUF]
        sem = gsem.at[s % _NBUF]
        for j in range(_SUP):
            pltpu.make_async_copy(
                table_hbm.at[idx_v.at[s * _SUP + j]],
                buf.at[pl.ds(j * _K, _K)],
                sem,
            ).wait()

    def out_dst(s):
        return out_hbm.at[pl.ds(out0 + s * _CHUNK, _CHUNK)]

    def issue_out(s):
        pltpu.async_copy(rows_v.at[s % _NBUF], out_dst(s), osem.at[s % _NBUF])

    def wait_out(s):
        pltpu.make_async_copy(
            rows_v.at[s % _NBUF], out_dst(s), osem.at[s % _NBUF]
        ).wait()

    fire(0)
    fire(1)

    def loop_body(s, c):
        wait_gathers(s)
        issue_out(s)

        @pl.when(s + 2 < _NSUP)
        def _():
            @pl.when(s >= 1)
            def _():
                wait_out(s - 1)

            fire(s + 2)

        return c

    lax.fori_loop(0, _NSUP, loop_body, 0)

    wait_out(_NSUP - 3)
    wait_out(_NSUP - 2)
    wait_out(_NSUP - 1)


@functools.partial(jax.jit, static_argnames=())
def kernel(inputs, tables):
    idx2d = inputs[..., 0].reshape(_TOTAL // _K, _K)
    table_flat = tables.reshape(_N_FIELDS * _VOCAB, _D)

    mesh = plsc.VectorSubcoreMesh(core_axis_name="c", subcore_axis_name="s")
    run = pl.kernel(
        _sc_body,
        out_type=jax.ShapeDtypeStruct((_TOTAL, _D), jnp.float32),
        mesh=mesh,
        scratch_types=[
            pltpu.VMEM((_NROWS, _K), jnp.int32),
            pltpu.VMEM((_NBUF, _CHUNK, _D), jnp.float32),
            pltpu.SemaphoreType.DMA((_NBUF,)),
            pltpu.SemaphoreType.DMA((_NBUF,)),
        ],
        compiler_params=pltpu.CompilerParams(use_tc_tiling_on_sc=False),
    )
    out_flat = run(idx2d, table_flat)
    return out_flat.reshape(_N_FIELDS, _B, _D)


# speedup vs baseline: 4.5905x; 4.4902x over previous
import functools

import jax
import jax.numpy as jnp
from jax import lax
from jax.experimental import pallas as pl
from jax.experimental.pallas import tpu as pltpu
from jax.experimental.pallas import tpu_sc as plsc

_N_FIELDS = 26
_VOCAB = 100000
_D = 32
_B = 16384

_NC = 2
_NS = 16
_NW = _NC * _NS
_HALF = _B // 2


def _sc_body(idx_hbm, tab_hbm, out_hbm, idx_v, col_v, out_v):
    wid = lax.axis_index("s") * _NC + lax.axis_index("c")

    def field_body(f, carry):
        r = f * _D + wid
        pltpu.sync_copy(tab_hbm.at[r], col_v)
        pltpu.sync_copy(idx_hbm.at[pl.ds(f * _B, _B)], idx_v)

        for h in range(2):
            def gather8(i, c, h=h):
                base = h * _HALF + i * 128
                for u in range(8):
                    sl = pl.ds(base + u * 16, 16)
                    osl = pl.ds(i * 128 + u * 16, 16)
                    out_v[osl] = plsc.load_gather(col_v, [idx_v[sl]])
                return c

            lax.fori_loop(0, _HALF // 128, gather8, 0)
            pltpu.sync_copy(out_v, out_hbm.at[r, pl.ds(h * _HALF, _HALF)])
        return carry

    lax.fori_loop(0, _N_FIELDS, field_body, 0)


@functools.partial(jax.jit, static_argnames=())
def kernel(inputs, tables):
    idxs = inputs.reshape(_N_FIELDS * _B)
    tab_t = jnp.transpose(tables, (0, 2, 1))
    tab2 = tab_t.reshape(_N_FIELDS * _D, _VOCAB)

    mesh = plsc.VectorSubcoreMesh(core_axis_name="c", subcore_axis_name="s")
    run = pl.kernel(
        _sc_body,
        out_type=jax.ShapeDtypeStruct((_N_FIELDS * _D, _B), jnp.float32),
        mesh=mesh,
        scratch_types=[
            pltpu.VMEM((_B,), jnp.int32),
            pltpu.VMEM((_VOCAB,), jnp.float32),
            pltpu.VMEM((_HALF,), jnp.float32),
        ],
        compiler_params=pltpu.CompilerParams(needs_layout_passes=False),
    )
    out_t = run(idxs, tab2)
    return out_t.reshape(_N_FIELDS, _D, _B).transpose(0, 2, 1)


# speedup vs baseline: 4.9737x; 1.0835x over previous
import functools

import jax
import jax.numpy as jnp
from jax import lax
from jax.experimental import pallas as pl
from jax.experimental.pallas import tpu as pltpu
from jax.experimental.pallas import tpu_sc as plsc

_N_FIELDS = 26
_VOCAB = 100000
_D = 32
_B = 16384

_NC = 2
_NS = 16
_NW = _NC * _NS
_HALF = _B // 2


_CCH = 4
_CW = _VOCAB // _CCH
_NH = 4
_Q = _B // _NH


def _sc_body(idx_hbm, tab_hbm, out_hbm, idx_v, col_v, out_v, csem, isem, osem):
    wid = lax.axis_index("s") * _NC + lax.axis_index("c")

    def col_copy(f):
        return [pltpu.make_async_copy(tab_hbm.at[f * _D + wid], col_v, csem)]

    def idx_copy(f, h, slot):
        return pltpu.make_async_copy(
            idx_hbm.at[pl.ds(f * _B + h * _Q, _Q)], idx_v.at[slot], isem
        )

    def out_copy(f, h, slot):
        return pltpu.make_async_copy(
            out_v.at[slot], out_hbm.at[f * _D + wid, pl.ds(h * _Q, _Q)], osem
        )

    for c in col_copy(0):
        c.start()
    idx_copy(0, 0, 0).start()

    def field_body(f, carry):
        for c in col_copy(f):
            c.wait()

        for h in range(_NH):
            slot = h % 2
            if h < _NH - 1:
                idx_copy(f, h + 1, 1 - slot).start()
            else:

                @pl.when(f + 1 < _N_FIELDS)
                def _(f=f, slot=slot):
                    idx_copy(f + 1, 0, 1 - slot).start()

            idx_copy(f, h, slot).wait()

            if h >= 2:
                out_copy(f, h, slot).wait()
            else:

                @pl.when(f >= 1)
                def _(f=f, h=h, slot=slot):
                    out_copy(f, h, slot).wait()

            def gather8(i, c, slot=slot):
                for u in range(8):
                    sl = pl.ds(i * 128 + u * 16, 16)
                    out_v[slot, sl] = plsc.load_gather(col_v, [idx_v[slot, sl]])
                return c

            lax.fori_loop(0, _Q // 128, gather8, 0)

            if h == _NH - 1:
                @pl.when(f + 1 < _N_FIELDS)
                def _(f=f):
                    for c in col_copy(f + 1):
                        c.start()

            out_copy(f, h, slot).start()
        return carry

    lax.fori_loop(0, _N_FIELDS, field_body, 0)

    out_copy(_N_FIELDS - 1, _NH - 2, 0).wait()
    out_copy(_N_FIELDS - 1, _NH - 1, 1).wait()


@functools.partial(jax.jit, static_argnames=())
def kernel(inputs, tables):
    idxs = inputs.reshape(_N_FIELDS * _B)
    tab_t = jnp.transpose(tables, (0, 2, 1))
    tab2 = tab_t.reshape(_N_FIELDS * _D, _VOCAB)

    mesh = plsc.VectorSubcoreMesh(core_axis_name="c", subcore_axis_name="s")
    run = pl.kernel(
        _sc_body,
        out_type=jax.ShapeDtypeStruct((_N_FIELDS * _D, _B), jnp.float32),
        mesh=mesh,
        scratch_types=[
            pltpu.VMEM((2, _Q), jnp.int32),
            pltpu.VMEM((_VOCAB,), jnp.float32),
            pltpu.VMEM((2, _Q), jnp.float32),
            pltpu.SemaphoreType.DMA,
            pltpu.SemaphoreType.DMA,
            pltpu.SemaphoreType.DMA,
        ],
        compiler_params=pltpu.CompilerParams(needs_layout_passes=False),
    )
    out_t = run(idxs, tab2)
    return out_t.reshape(_N_FIELDS, _D, _B).transpose(0, 2, 1)


# speedup vs baseline: 4.9924x; 1.0038x over previous
import functools

import jax
import jax.numpy as jnp
from jax import lax
from jax.experimental import pallas as pl
from jax.experimental.pallas import tpu as pltpu
from jax.experimental.pallas import tpu_sc as plsc

_N_FIELDS = 26
_VOCAB = 100000
_D = 32
_B = 16384

_NC = 2
_NS = 16
_NW = _NC * _NS
_HALF = _B // 2


_CCH = 4
_CW = _VOCAB // _CCH
_NH = 4
_Q = _B // _NH


def _sc_body(idx_hbm, tab_hbm, out_hbm, idx_v, col_v, out_v, csem, isem, osem):
    wid = lax.axis_index("s") * _NC + lax.axis_index("c")

    def col_copy(f):
        return [pltpu.make_async_copy(tab_hbm.at[f * _D + wid], col_v, csem)]

    def idx_copy(f, h, slot):
        return pltpu.make_async_copy(
            idx_hbm.at[pl.ds(f * _B + h * _Q, _Q)], idx_v.at[slot], isem
        )

    def out_copy(f, h, slot):
        return pltpu.make_async_copy(
            out_v.at[slot], out_hbm.at[f * _D + wid, pl.ds(h * _Q, _Q)], osem
        )

    for c in col_copy(0):
        c.start()
    idx_copy(0, 0, 0).start()

    def field_body(f, carry):
        for c in col_copy(f):
            c.wait()

        for h in range(_NH):
            slot = h % 2
            if h < _NH - 1:
                idx_copy(f, h + 1, 1 - slot).start()
            else:

                @pl.when(f + 1 < _N_FIELDS)
                def _(f=f, slot=slot):
                    idx_copy(f + 1, 0, 1 - slot).start()

            idx_copy(f, h, slot).wait()

            if h >= 2:
                out_copy(f, h, slot).wait()
            else:

                @pl.when(f >= 1)
                def _(f=f, h=h, slot=slot):
                    out_copy(f, h, slot).wait()

            def gather16(i, c, slot=slot):
                for u in range(16):
                    sl = pl.ds(i * 256 + u * 16, 16)
                    out_v[slot, sl] = plsc.load_gather(col_v, [idx_v[slot, sl]])
                return c

            lax.fori_loop(0, _Q // 256, gather16, 0)

            if h == _NH - 1:
                @pl.when(f + 1 < _N_FIELDS)
                def _(f=f):
                    for c in col_copy(f + 1):
                        c.start()

            out_copy(f, h, slot).start()
        return carry

    lax.fori_loop(0, _N_FIELDS, field_body, 0)

    out_copy(_N_FIELDS - 1, _NH - 2, 0).wait()
    out_copy(_N_FIELDS - 1, _NH - 1, 1).wait()


@functools.partial(jax.jit, static_argnames=())
def kernel(inputs, tables):
    idxs = inputs.reshape(_N_FIELDS * _B)
    tab_t = jnp.transpose(tables, (0, 2, 1))
    tab2 = tab_t.reshape(_N_FIELDS * _D, _VOCAB)

    mesh = plsc.VectorSubcoreMesh(core_axis_name="c", subcore_axis_name="s")
    run = pl.kernel(
        _sc_body,
        out_type=jax.ShapeDtypeStruct((_N_FIELDS * _D, _B), jnp.float32),
        mesh=mesh,
        scratch_types=[
            pltpu.VMEM((2, _Q), jnp.int32),
            pltpu.VMEM((_VOCAB,), jnp.float32),
            pltpu.VMEM((2, _Q), jnp.float32),
            pltpu.SemaphoreType.DMA,
            pltpu.SemaphoreType.DMA,
            pltpu.SemaphoreType.DMA,
        ],
        compiler_params=pltpu.CompilerParams(needs_layout_passes=False),
    )
    out_t = run(idxs, tab2)
    return out_t.reshape(_N_FIELDS, _D, _B).transpose(0, 2, 1)


# speedup vs baseline: 6.3338x; 1.2687x over previous
import functools

import jax
import jax.numpy as jnp
from jax import lax
from jax.experimental import pallas as pl
from jax.experimental.pallas import tpu as pltpu
from jax.experimental.pallas import tpu_sc as plsc

_N_FIELDS = 26
_VOCAB = 100000
_D = 32
_B = 16384

_NC = 2
_NS = 16
_NW = _NC * _NS
_HALF = _B // 2


_CCH = 4
_CW = _VOCAB // _CCH
_NH = 4
_Q = _B // _NH


def _sc_body(idx_hbm, tab_hbm, out_hbm, idx_v, col_v, out_v, csem, isem, osem):
    wid = lax.axis_index("s") * _NC + lax.axis_index("c")

    def col_copy(f):
        return [pltpu.make_async_copy(tab_hbm.at[f * _D + wid], col_v, csem)]

    def idx_copy(f, h, slot):
        return pltpu.make_async_copy(
            idx_hbm.at[pl.ds(f * _B + h * _Q, _Q)], idx_v.at[slot], isem
        )

    def out_copy(f, h, slot):
        return pltpu.make_async_copy(
            out_v.at[slot], out_hbm.at[f * _D + wid, pl.ds(h * _Q, _Q)], osem
        )

    for c in col_copy(0):
        c.start()
    idx_copy(0, 0, 0).start()

    def field_body(f, carry):
        for c in col_copy(f):
            c.wait()

        for h in range(_NH):
            slot = h % 2
            if h < _NH - 1:
                idx_copy(f, h + 1, 1 - slot).start()
            else:

                @pl.when(f + 1 < _N_FIELDS)
                def _(f=f, slot=slot):
                    idx_copy(f + 1, 0, 1 - slot).start()

            idx_copy(f, h, slot).wait()

            if h >= 2:
                out_copy(f, h, slot).wait()
            else:

                @pl.when(f >= 1)
                def _(f=f, h=h, slot=slot):
                    out_copy(f, h, slot).wait()

            @plsc.parallel_loop(0, _Q, step=16, unroll=8)
            def _(i, slot=slot):
                sl = pl.ds(i, 16)
                out_v[slot, sl] = plsc.load_gather(col_v, [idx_v[slot, sl]])

            if h == _NH - 1:
                @pl.when(f + 1 < _N_FIELDS)
                def _(f=f):
                    for c in col_copy(f + 1):
                        c.start()

            out_copy(f, h, slot).start()
        return carry

    lax.fori_loop(0, _N_FIELDS, field_body, 0)

    out_copy(_N_FIELDS - 1, _NH - 2, 0).wait()
    out_copy(_N_FIELDS - 1, _NH - 1, 1).wait()


@functools.partial(jax.jit, static_argnames=())
def kernel(inputs, tables):
    idxs = inputs.reshape(_N_FIELDS * _B)
    tab_t = jnp.transpose(tables, (0, 2, 1))
    tab2 = tab_t.reshape(_N_FIELDS * _D, _VOCAB)

    mesh = plsc.VectorSubcoreMesh(core_axis_name="c", subcore_axis_name="s")
    run = pl.kernel(
        _sc_body,
        out_type=jax.ShapeDtypeStruct((_N_FIELDS * _D, _B), jnp.float32),
        mesh=mesh,
        scratch_types=[
            pltpu.VMEM((2, _Q), jnp.int32),
            pltpu.VMEM((_VOCAB,), jnp.float32),
            pltpu.VMEM((2, _Q), jnp.float32),
            pltpu.SemaphoreType.DMA,
            pltpu.SemaphoreType.DMA,
            pltpu.SemaphoreType.DMA,
        ],
        compiler_params=pltpu.CompilerParams(needs_layout_passes=False),
    )
    out_t = run(idxs, tab2)
    return out_t.reshape(_N_FIELDS, _D, _B).transpose(0, 2, 1)


# speedup vs baseline: 6.3479x; 1.0022x over previous
import functools

import jax
import jax.numpy as jnp
from jax import lax
from jax.experimental import pallas as pl
from jax.experimental.pallas import tpu as pltpu
from jax.experimental.pallas import tpu_sc as plsc

_N_FIELDS = 26
_VOCAB = 100000
_D = 32
_B = 16384

_NC = 2
_NS = 16
_NW = _NC * _NS
_HALF = _B // 2


_CCH = 4
_CW = _VOCAB // _CCH
_NH = 4
_Q = _B // _NH


def _sc_body(idx_hbm, tab_hbm, out_hbm, idx_v, col_v, out_v, csem, isem, osem):
    wid = lax.axis_index("s") * _NC + lax.axis_index("c")

    def col_copy(f):
        return [pltpu.make_async_copy(tab_hbm.at[f * _D + wid], col_v, csem)]

    def idx_copy(f, h, slot):
        return pltpu.make_async_copy(
            idx_hbm.at[pl.ds(f * _B + h * _Q, _Q)], idx_v.at[slot], isem
        )

    def out_copy(f, h, slot):
        return pltpu.make_async_copy(
            out_v.at[slot], out_hbm.at[f * _D + wid, pl.ds(h * _Q, _Q)], osem
        )

    for c in col_copy(0):
        c.start()
    idx_copy(0, 0, 0).start()

    def field_body(f, carry):
        for c in col_copy(f):
            c.wait()

        for h in range(_NH):
            slot = h % 2
            if h < _NH - 1:
                idx_copy(f, h + 1, 1 - slot).start()
            else:

                @pl.when(f + 1 < _N_FIELDS)
                def _(f=f, slot=slot):
                    idx_copy(f + 1, 0, 1 - slot).start()

            idx_copy(f, h, slot).wait()

            if h >= 2:
                out_copy(f, h, slot).wait()
            else:

                @pl.when(f >= 1)
                def _(f=f, h=h, slot=slot):
                    out_copy(f, h, slot).wait()

            @plsc.parallel_loop(0, _Q, step=16, unroll=16)
            def _(i, slot=slot):
                sl = pl.ds(i, 16)
                out_v[slot, sl] = plsc.load_gather(col_v, [idx_v[slot, sl]])

            if h == _NH - 1:
                @pl.when(f + 1 < _N_FIELDS)
                def _(f=f):
                    for c in col_copy(f + 1):
                        c.start()

            out_copy(f, h, slot).start()
        return carry

    lax.fori_loop(0, _N_FIELDS, field_body, 0)

    out_copy(_N_FIELDS - 1, _NH - 2, 0).wait()
    out_copy(_N_FIELDS - 1, _NH - 1, 1).wait()


@functools.partial(jax.jit, static_argnames=())
def kernel(inputs, tables):
    idxs = inputs.reshape(_N_FIELDS * _B)
    tab_t = jnp.transpose(tables, (0, 2, 1))
    tab2 = tab_t.reshape(_N_FIELDS * _D, _VOCAB)

    mesh = plsc.VectorSubcoreMesh(core_axis_name="c", subcore_axis_name="s")
    run = pl.kernel(
        _sc_body,
        out_type=jax.ShapeDtypeStruct((_N_FIELDS * _D, _B), jnp.float32),
        mesh=mesh,
        scratch_types=[
            pltpu.VMEM((2, _Q), jnp.int32),
            pltpu.VMEM((_VOCAB,), jnp.float32),
            pltpu.VMEM((2, _Q), jnp.float32),
            pltpu.SemaphoreType.DMA,
            pltpu.SemaphoreType.DMA,
            pltpu.SemaphoreType.DMA,
        ],
        compiler_params=pltpu.CompilerParams(needs_layout_passes=False),
    )
    out_t = run(idxs, tab2)
    return out_t.reshape(_N_FIELDS, _D, _B).transpose(0, 2, 1)
